# baseline (speedup 1.0000x reference)
import functools

import jax
import jax.numpy as jnp
import numpy as np
from jax import lax
from jax.experimental import pallas as pl
from jax.experimental.pallas import tpu as pltpu
from jax.experimental.pallas import tpu_sc as plsc

D = 64
S = 200
B = 1024

NC, NS, L = 2, 16, 16
NW = NC * NS
BB = 128
NBB = B // BB
NBLK = S * NBB
BLK_PER_W = NBLK // NW
NBUF = 5


def _pe_table() -> np.ndarray:
    position = np.arange(S)[:, np.newaxis]
    div_term = np.exp(np.arange(0, D, 2) * (-np.log(10000.0) / D))
    pe = np.zeros((S, D), dtype=np.float32)
    pe[:, 0::2] = np.sin(position * div_term)
    pe[:, 1::2] = np.cos(position * div_term)
    return pe


_PE = _pe_table()

_mesh = plsc.VectorSubcoreMesh(core_axis_name="c", subcore_axis_name="s")


@functools.partial(
    pl.kernel,
    out_type=jax.ShapeDtypeStruct((S, D // 8, NBB, 8, BB), jnp.float32),
    mesh=_mesh,
    scratch_types=[
        pltpu.VMEM((BLK_PER_W, BB), jnp.int32),
        pltpu.VMEM((S * D // BB + 4, BB), jnp.float32),
        pltpu.VMEM((NBUF, BB, D), jnp.float32),
        pltpu.VMEM((NBUF, D, BB + 1), jnp.float32),
    ] + [pltpu.SemaphoreType.DMA] * (2 * NBUF),
    compiler_params=pltpu.CompilerParams(
        use_tc_tiling_on_sc=False, needs_layout_passes=False),
)
def _encode(emb_hbm, x_hbm, pe_hbm, out_hbm, idx_v, pe_v, gbuf, tbuf, *sems):
    wid = lax.axis_index("s") * NC + lax.axis_index("c")
    r0 = wid * BLK_PER_W
    pltpu.sync_copy(x_hbm.at[pl.ds(r0, BLK_PER_W)], idx_v)
    pltpu.sync_copy(pe_hbm, pe_v)

    gsems = sems[:NBUF]
    ssems = sems[NBUF:]

    def issue_gather(i, slot):
        pltpu.async_copy(
            emb_hbm.at[idx_v.at[i]], gbuf.at[slot], gsems[slot])

    def wait_gather(i, slot):
        pltpu.make_async_copy(
            emb_hbm.at[idx_v.at[i]], gbuf.at[slot], gsems[slot]).wait()

    def store_rows(i, slot, wait):
        r = r0 + i
        s = (r // (8 * NBB)) * 8 + r % 8
        b_hi = (r // 8) % NBB
        for d_hi in range(D // 8):
            src = tbuf.at[slot, pl.ds(d_hi * 8, 8), pl.ds(0, BB)]
            dst = out_hbm.at[s, d_hi, b_hi]
            if wait:
                pltpu.make_async_copy(src, dst, ssems[slot]).wait()
            else:
                pltpu.async_copy(src, dst, ssems[slot])

    for b in range(NBUF - 1):
        issue_gather(b, b)

    riq = [16 * q + lax.iota(jnp.int32, L) for q in range(D // L)]

    def group(g, _):
        for t in range(NBUF):
            i = NBUF * g + t
            r = r0 + i
            s = (r // (8 * NBB)) * 8 + r % 8

            @pl.when(i + NBUF - 1 < BLK_PER_W)
            def _():
                issue_gather(i + NBUF - 1, (t + NBUF - 1) % NBUF)

            wait_gather(i, t)

            @pl.when(i >= NBUF)
            def _():
                store_rows(i, t, wait=True)

            src = gbuf.at[t]
            dst = tbuf.at[t]
            prow, pcol = s // 2, (s % 2) * D
            pe4 = [pe_v[prow, pl.ds(pcol + q * L, L)] for q in range(D // L)]

            @plsc.parallel_loop(0, BB, step=1, unroll=4)
            def _transpose(i):
                ci = jnp.full((L,), i, jnp.int32)
                for q in range(D // L):
                    v = src[i, pl.ds(q * L, L)] + pe4[q]
                    plsc.store_scatter(dst, [riq[q], ci], v)

            store_rows(i, t, wait=False)
        return 0

    lax.fori_loop(0, BLK_PER_W // NBUF, group, 0)
    for t in range(NBUF):
        store_rows(BLK_PER_W - NBUF + t, t, wait=True)


def kernel(x, emb):
    x4 = x.T.reshape(S // 8, 8, NBB, BB).transpose(0, 2, 1, 3)
    xr = x4.reshape(S // 8 * NBB * 8, BB)
    pe_pack = np.zeros((S * D // BB + 4, BB), np.float32)
    pe_pack.reshape(-1)[: S * D] = _PE.reshape(-1)
    never = x[0, 0] < jnp.int32(-1)
    pe_rt = jnp.where(never, jnp.float32(0), jnp.asarray(pe_pack))
    emb_p = jnp.pad(emb, ((0, 7), (0, D))).reshape(2 * (100001 + 7), D)
    out5 = _encode(emb_p, xr * 2, pe_rt)
    return out5.transpose(2, 4, 0, 1, 3).reshape(B, S, D)
